# baseline (speedup 1.0000x reference)
import functools

import jax
import jax.numpy as jnp
from jax import lax
from jax.experimental import pallas as pl
from jax.experimental.pallas import tpu as pltpu
from jax.experimental.pallas import tpu_sc as plsc

N_DIM = 64
N_ROWS = 100000
BATCH = 16384
EPS = 1e-5

L = 16
NC, NS = 2, 16
NW = NC * NS
BPW = BATCH // NW
CHUNK = 128
NCHUNK = BPW // CHUNK
GROUPS = BPW // L
GPC = GROUPS // NCHUNK

LN2 = 0.6931471805599453

PIECE = 512
NFULL = 195
PPT = 7
TAILS = ((NFULL * PIECE, 128),)
TCOV = NFULL * PIECE + 128


def _tr_body(uv_t, iv_t, uflat, iflat, in_v, out_v, sems):
    wid = lax.axis_index("s") * NC + lax.axis_index("c")

    lane = lax.iota(jnp.int32, L)
    scat_base = lane * N_DIM

    def shuffle_k(k, _):
        kbase = k * (L * N_DIM)
        for d in range(N_DIM):
            x = in_v[d, pl.ds(k * L, L)]
            plsc.store_scatter(out_v, [kbase + d + scat_base], x)
        return 0

    for table_t, flat_out in ((uv_t, uflat), (iv_t, iflat)):
        for p in range(PPT):
            piece_id = jnp.minimum(wid * PPT + p, NFULL - 1)
            a = pl.multiple_of(piece_id * PIECE, 128)
            pltpu.async_copy(
                table_t.at[:, pl.ds(a, PIECE)], in_v, sems.at[0]).wait()
            lax.fori_loop(0, PIECE // L, shuffle_k, 0)
            pltpu.async_copy(
                out_v, flat_out.at[pl.ds(a * N_DIM, PIECE * N_DIM)],
                sems.at[1]).wait()
        for ta, tw in TAILS:
            pltpu.async_copy(
                table_t.at[:, pl.ds(ta, tw)], in_v.at[:, pl.ds(0, tw)],
                sems.at[0]).wait()
            lax.fori_loop(0, tw // L, shuffle_k, 0)
            pltpu.async_copy(
                out_v.at[pl.ds(0, tw * N_DIM)],
                flat_out.at[pl.ds(ta * N_DIM, tw * N_DIM)],
                sems.at[1]).wait()


def _sc_body(u_hbm, i_hbm, uvect_hbm, ubias_hbm, ivect_hbm, ibias_hbm, gb_hbm,
             out_hbm,
             uidx_v, iidx_v, urows_v, irows_v, ubias_v, ibias_v,
             gb_v, out_v, sems):
    wid = lax.axis_index("s") * NC + lax.axis_index("c")
    base = wid * BPW

    pltpu.sync_copy(gb_hbm, gb_v)
    pltpu.sync_copy(u_hbm.at[pl.ds(base, BPW)], uidx_v)
    pltpu.sync_copy(i_hbm.at[pl.ds(base, BPW)], iidx_v)

    copies = [[] for _ in range(NCHUNK)]
    for j in range(NCHUNK):
        sl = pl.ds(j * CHUNK, CHUNK)
        copies[j].append(
            pltpu.async_copy(uvect_hbm.at[uidx_v.at[sl]], urows_v.at[sl], sems.at[j]))
        copies[j].append(
            pltpu.async_copy(ivect_hbm.at[iidx_v.at[sl]], irows_v.at[sl], sems.at[j]))
        copies[j].append(
            pltpu.async_copy(ubias_hbm.at[uidx_v.at[sl]], ubias_v.at[sl], sems.at[j]))
        copies[j].append(
            pltpu.async_copy(ibias_hbm.at[iidx_v.at[sl]], ibias_v.at[sl], sems.at[j]))

    lane = lax.iota(jnp.int32, L)
    zf = jnp.zeros((L,), jnp.float32)
    gb = gb_v[...]

    def group_body(g, _):
        rows = g * L + lane
        sq0, nu0, nv0 = zf, zf, zf
        sq1, nu1, nv1 = zf, zf, zf
        for d in range(N_DIM):
            dsplat = jnp.full((L,), d, jnp.int32)
            xu = plsc.load_gather(urows_v, [rows, dsplat])
            xi = plsc.load_gather(irows_v, [rows, dsplat])
            diff = xu - xi
            if d % 2 == 0:
                sq0 = sq0 + diff * diff
                nu0 = nu0 + xu * xu
                nv0 = nv0 + xi * xi
            else:
                sq1 = sq1 + diff * diff
                nu1 = nu1 + xu * xu
                nv1 = nv1 + xi * xi
        sq = sq0 + sq1
        nu = nu0 + nu1
        nv = nv0 + nv1
        arg = 1.0 + 2.0 * sq / ((1.0 - nu) * (1.0 - nv) + EPS)
        a = jnp.maximum(arg, 1.0 + EPS)
        x = a * a - 1.0
        yi = 0x5F3759DF - lax.shift_right_logical(plsc.bitcast(x, jnp.int32), 1)
        y = plsc.bitcast(yi, jnp.float32)
        y = y * (1.5 - 0.5 * x * y * y)
        y = y * (1.5 - 0.5 * x * y * y)
        y = y * (1.5 - 0.5 * x * y * y)
        z = a + x * y
        zb = plsc.bitcast(z, jnp.int32)
        e = lax.shift_right_logical(zb, 23) - 127
        m = plsc.bitcast((zb & 0x007FFFFF) | 0x3F800000, jnp.float32)
        t = (m - 1.0) / (m + 1.0)
        t2 = t * t
        lnm = 2.0 * t * (1.0 + t2 * (1.0 / 3.0 + t2 * (0.2 + t2 * (1.0 / 7.0))))
        dist = LN2 * e.astype(jnp.float32) + lnm
        sl = pl.ds(g * L, L)
        out_v[sl] = gb + ubias_v[sl] + ibias_v[sl] + dist
        return 0

    for c in range(NCHUNK):
        for cp in copies[c]:
            cp.wait()
        lax.fori_loop(c * GPC, (c + 1) * GPC, group_body, 0)

    pltpu.sync_copy(out_v, out_hbm.at[pl.ds(base, BPW)])


_transpose_kernel = functools.partial(
    pl.kernel,
    out_type=[
        jax.ShapeDtypeStruct((N_ROWS * N_DIM,), jnp.float32),
        jax.ShapeDtypeStruct((N_ROWS * N_DIM,), jnp.float32),
    ],
    mesh=plsc.VectorSubcoreMesh(core_axis_name="c", subcore_axis_name="s"),
    compiler_params=pltpu.CompilerParams(needs_layout_passes=False),
    scratch_types=[
        pltpu.VMEM((N_DIM, PIECE), jnp.float32),
        pltpu.VMEM((PIECE * N_DIM,), jnp.float32),
        pltpu.SemaphoreType.DMA((2,)),
    ],
)(_tr_body)


_sc_kernel = functools.partial(
    pl.kernel,
    out_type=jax.ShapeDtypeStruct((BATCH,), jnp.float32),
    mesh=plsc.VectorSubcoreMesh(core_axis_name="c", subcore_axis_name="s"),
    compiler_params=pltpu.CompilerParams(
        needs_layout_passes=False, use_tc_tiling_on_sc=False
    ),
    scratch_types=[
        pltpu.VMEM((BPW,), jnp.int32),
        pltpu.VMEM((BPW,), jnp.int32),
        pltpu.VMEM((BPW, N_DIM), jnp.float32),
        pltpu.VMEM((BPW, N_DIM), jnp.float32),
        pltpu.VMEM((BPW,), jnp.float32),
        pltpu.VMEM((BPW,), jnp.float32),
        pltpu.VMEM((L,), jnp.float32),
        pltpu.VMEM((BPW,), jnp.float32),
        pltpu.SemaphoreType.DMA((NCHUNK,)),
    ],
)(_sc_body)


@jax.jit
def _impl(u, i, user_vect, user_bias, item_vect, item_bias, glob_bias):
    uflat, iflat = _transpose_kernel(user_vect.T, item_vect.T)
    uflat = uflat.at[TCOV * N_DIM:].set(user_vect[TCOV:].reshape(-1))
    iflat = iflat.at[TCOV * N_DIM:].set(item_vect[TCOV:].reshape(-1))
    return _sc_kernel(
        u.astype(jnp.int32), i.astype(jnp.int32),
        uflat.reshape(N_ROWS, N_DIM), user_bias.reshape(-1),
        iflat.reshape(N_ROWS, N_DIM), item_bias.reshape(-1),
        jnp.broadcast_to(glob_bias.reshape(1), (L,)),
    )


def kernel(u, i, user_vect, user_bias, item_vect, item_bias, glob_bias):
    return _impl(u, i, user_vect, user_bias, item_vect, item_bias, glob_bias)


# speedup vs baseline: 2.3515x; 2.3515x over previous
import functools

import jax
import jax.numpy as jnp
from jax import lax
from jax.experimental import pallas as pl
from jax.experimental.pallas import tpu as pltpu
from jax.experimental.pallas import tpu_sc as plsc

N_DIM = 64
BATCH = 16384
EPS = 1e-5

L = 16
NC, NS = 2, 16
NW = NC * NS
BPW = BATCH // NW
CHUNK = 128
NCHUNK = BPW // CHUNK
GROUPS = BPW // L
GPC = GROUPS // NCHUNK
PAIR = 2 * N_DIM

LN2 = 0.6931471805599453


def _item_body(i_hbm, ivect_hbm, ibias_hbm,
               irows_out, ibias_out,
               iidx_v, ipidx_v, irows_v, ibias_v, sems):
    wid = lax.axis_index("s") * NC + lax.axis_index("c")
    base = wid * BPW

    pltpu.sync_copy(i_hbm.at[pl.ds(base, BPW)], iidx_v)
    for t in range(BPW // L):
        sl = pl.ds(t * L, L)
        ipidx_v[sl] = lax.shift_right_logical(iidx_v[sl], 1)

    def fire(c):
        sl = pl.ds(c * CHUNK, CHUNK)
        return [
            pltpu.async_copy(ivect_hbm.at[ipidx_v.at[sl]], irows_v.at[c % 2],
                             sems.at[c]),
            pltpu.async_copy(ibias_hbm.at[iidx_v.at[sl]], ibias_v.at[sl],
                             sems.at[c]),
        ]

    inflight = {0: fire(0), 1: fire(1)}
    stores = []
    for c in range(NCHUNK):
        for cp in inflight.pop(c):
            cp.wait()
        stores.append(pltpu.async_copy(
            irows_v.at[c % 2],
            irows_out.at[pl.ds(base + c * CHUNK, CHUNK)],
            sems.at[NCHUNK + (c % 2)]))
        if c + 2 < NCHUNK:
            stores[c].wait()
            inflight[c + 2] = fire(c + 2)
    for s in stores[-2:]:
        s.wait()
    pltpu.sync_copy(ibias_v, ibias_out.at[pl.ds(base, BPW)])


def _sc_body(u_hbm, i_hbm, uvect_hbm, ubias_hbm, irows_hbm, ibias_vals_hbm,
             gb_hbm, out_hbm,
             uidx_v, iidx_v, upidx_v,
             urows_v, irows_v, ubias_v, ibias_v,
             gb_v, out_v, sems):
    wid = lax.axis_index("s") * NC + lax.axis_index("c")
    base = wid * BPW

    pltpu.sync_copy(gb_hbm, gb_v)
    pltpu.sync_copy(u_hbm.at[pl.ds(base, BPW)], uidx_v)
    pltpu.sync_copy(i_hbm.at[pl.ds(base, BPW)], iidx_v)
    pltpu.sync_copy(ibias_vals_hbm.at[pl.ds(base, BPW)], ibias_v)

    for t in range(BPW // L):
        sl = pl.ds(t * L, L)
        upidx_v[sl] = lax.shift_right_logical(uidx_v[sl], 1)

    def fire(c):
        sl = pl.ds(c * CHUNK, CHUNK)
        b = c % 2
        return [
            pltpu.async_copy(uvect_hbm.at[upidx_v.at[sl]], urows_v.at[b], sems.at[c]),
            pltpu.async_copy(irows_hbm.at[pl.ds(base + c * CHUNK, CHUNK)],
                             irows_v.at[b], sems.at[c]),
            pltpu.async_copy(ubias_hbm.at[uidx_v.at[sl]], ubias_v.at[sl], sems.at[c]),
        ]

    lane = lax.iota(jnp.int32, L)
    zf = jnp.zeros((L,), jnp.float32)
    gb = gb_v[...]

    def compute_chunk(c):
        b = c % 2
        ubuf = urows_v.at[b]
        ibuf = irows_v.at[b]

        def group_body(gg, _):
            rows = gg * L + lane
            gsl = pl.ds(c * CHUNK + gg * L, L)
            pu = (uidx_v[gsl] & 1) * N_DIM
            pi = (iidx_v[gsl] & 1) * N_DIM
            sq0, nu0, nv0 = zf, zf, zf
            sq1, nu1, nv1 = zf, zf, zf
            for d in range(N_DIM):
                xu = plsc.load_gather(ubuf, [rows, pu + d])
                xi = plsc.load_gather(ibuf, [rows, pi + d])
                diff = xu - xi
                if d % 2 == 0:
                    sq0 = sq0 + diff * diff
                    nu0 = nu0 + xu * xu
                    nv0 = nv0 + xi * xi
                else:
                    sq1 = sq1 + diff * diff
                    nu1 = nu1 + xu * xu
                    nv1 = nv1 + xi * xi
            sq = sq0 + sq1
            nu = nu0 + nu1
            nv = nv0 + nv1
            arg = 1.0 + 2.0 * sq / ((1.0 - nu) * (1.0 - nv) + EPS)
            a = jnp.maximum(arg, 1.0 + EPS)
            x = a * a - 1.0
            yi = 0x5F3759DF - lax.shift_right_logical(plsc.bitcast(x, jnp.int32), 1)
            y = plsc.bitcast(yi, jnp.float32)
            y = y * (1.5 - 0.5 * x * y * y)
            y = y * (1.5 - 0.5 * x * y * y)
            y = y * (1.5 - 0.5 * x * y * y)
            z = a + x * y
            zb = plsc.bitcast(z, jnp.int32)
            e = lax.shift_right_logical(zb, 23) - 127
            m = plsc.bitcast((zb & 0x007FFFFF) | 0x3F800000, jnp.float32)
            t = (m - 1.0) / (m + 1.0)
            t2 = t * t
            lnm = 2.0 * t * (1.0 + t2 * (1.0 / 3.0 + t2 * (0.2 + t2 * (1.0 / 7.0))))
            dist = LN2 * e.astype(jnp.float32) + lnm
            out_v[gsl] = gb + ubias_v[gsl] + ibias_v[gsl] + dist
            return 0

        lax.fori_loop(0, GPC, group_body, 0)

    inflight = {0: fire(0), 1: fire(1)}
    for c in range(NCHUNK):
        for cp in inflight.pop(c):
            cp.wait()
        compute_chunk(c)
        if c + 2 < NCHUNK:
            inflight[c + 2] = fire(c + 2)

    pltpu.sync_copy(out_v, out_hbm.at[pl.ds(base, BPW)])


_item_kernel = functools.partial(
    pl.kernel,
    out_type=[
        jax.ShapeDtypeStruct((BATCH, PAIR), jnp.float32),
        jax.ShapeDtypeStruct((BATCH,), jnp.float32),
    ],
    mesh=plsc.VectorSubcoreMesh(core_axis_name="c", subcore_axis_name="s"),
    compiler_params=pltpu.CompilerParams(needs_layout_passes=False),
    scratch_types=[
        pltpu.VMEM((BPW,), jnp.int32),
        pltpu.VMEM((BPW,), jnp.int32),
        pltpu.VMEM((2, CHUNK, PAIR), jnp.float32),
        pltpu.VMEM((BPW,), jnp.float32),
        pltpu.SemaphoreType.DMA((NCHUNK + 2,)),
    ],
)(_item_body)


_sc_kernel = functools.partial(
    pl.kernel,
    out_type=jax.ShapeDtypeStruct((BATCH,), jnp.float32),
    mesh=plsc.VectorSubcoreMesh(core_axis_name="c", subcore_axis_name="s"),
    compiler_params=pltpu.CompilerParams(needs_layout_passes=False),
    scratch_types=[
        pltpu.VMEM((BPW,), jnp.int32),
        pltpu.VMEM((BPW,), jnp.int32),
        pltpu.VMEM((BPW,), jnp.int32),
        pltpu.VMEM((2, CHUNK, PAIR), jnp.float32),
        pltpu.VMEM((2, CHUNK, PAIR), jnp.float32),
        pltpu.VMEM((BPW,), jnp.float32),
        pltpu.VMEM((BPW,), jnp.float32),
        pltpu.VMEM((L,), jnp.float32),
        pltpu.VMEM((BPW,), jnp.float32),
        pltpu.SemaphoreType.DMA((NCHUNK,)),
    ],
)(_sc_body)


@jax.jit
def _impl(u, i, user_vect, user_bias, item_vect, item_bias, glob_bias):
    i32 = i.astype(jnp.int32)
    irows, ibias_vals = _item_kernel(
        i32, item_vect.reshape(-1, PAIR), item_bias.reshape(-1))
    return _sc_kernel(
        u.astype(jnp.int32), i32,
        user_vect.reshape(-1, PAIR), user_bias.reshape(-1),
        irows, ibias_vals,
        jnp.broadcast_to(glob_bias.reshape(1), (L,)),
    )


def kernel(u, i, user_vect, user_bias, item_vect, item_bias, glob_bias):
    return _impl(u, i, user_vect, user_bias, item_vect, item_bias, glob_bias)


# speedup vs baseline: 2.4491x; 1.0415x over previous
import functools

import jax
import jax.numpy as jnp
from jax import lax
from jax.experimental import pallas as pl
from jax.experimental.pallas import tpu as pltpu
from jax.experimental.pallas import tpu_sc as plsc

N_DIM = 64
BATCH = 16384
EPS = 1e-5

L = 16
NC, NS = 2, 16
NW = NC * NS
BPW = BATCH // NW
CHUNK = 128
NCHUNK = BPW // CHUNK
GROUPS = BPW // L
GPC = GROUPS // NCHUNK
PAIR = 2 * N_DIM

LN2 = 0.6931471805599453


def _sc_body(u_hbm, i_hbm, uvect_hbm, ubias_hbm, ivect_hbm, ibias_hbm, gb_hbm,
             out_hbm,
             uidx_v, iidx_v,
             urows_v, irows_v, ubias_v, ibias_v,
             gb_v, out_v, sems):
    wid = lax.axis_index("s") * NC + lax.axis_index("c")
    base = wid * BPW

    pltpu.sync_copy(gb_hbm, gb_v)
    pltpu.sync_copy(u_hbm.at[pl.ds(base, BPW)], uidx_v)
    pltpu.sync_copy(i_hbm.at[pl.ds(base, BPW)], iidx_v)

    def fire(c):
        sl = pl.ds(c * CHUNK, CHUNK)
        b = c % 2
        return [
            pltpu.async_copy(uvect_hbm.at[uidx_v.at[sl]], urows_v.at[b], sems.at[c]),
            pltpu.async_copy(ivect_hbm.at[iidx_v.at[sl]], irows_v.at[b], sems.at[c]),
            pltpu.async_copy(ubias_hbm.at[uidx_v.at[sl]], ubias_v.at[sl], sems.at[c]),
            pltpu.async_copy(ibias_hbm.at[iidx_v.at[sl]], ibias_v.at[sl], sems.at[c]),
        ]

    lane = lax.iota(jnp.int32, L)
    zf = jnp.zeros((L,), jnp.float32)
    gb = gb_v[...]

    def compute_chunk(c):
        b = c % 2
        ubuf = urows_v.at[b]
        ibuf = irows_v.at[b]

        def group_body(gg, _):
            rows = gg * L + lane
            gsl = pl.ds(c * CHUNK + gg * L, L)
            sq0, nu0, nv0 = zf, zf, zf
            sq1, nu1, nv1 = zf, zf, zf
            for d in range(N_DIM):
                dsplat = jnp.full((L,), d, jnp.int32)
                xu = plsc.load_gather(ubuf, [rows, dsplat])
                xi = plsc.load_gather(ibuf, [rows, dsplat])
                diff = xu - xi
                if d % 2 == 0:
                    sq0 = sq0 + diff * diff
                    nu0 = nu0 + xu * xu
                    nv0 = nv0 + xi * xi
                else:
                    sq1 = sq1 + diff * diff
                    nu1 = nu1 + xu * xu
                    nv1 = nv1 + xi * xi
            sq = sq0 + sq1
            nu = nu0 + nu1
            nv = nv0 + nv1
            arg = 1.0 + 2.0 * sq / ((1.0 - nu) * (1.0 - nv) + EPS)
            a = jnp.maximum(arg, 1.0 + EPS)
            x = a * a - 1.0
            yi = 0x5F3759DF - lax.shift_right_logical(plsc.bitcast(x, jnp.int32), 1)
            y = plsc.bitcast(yi, jnp.float32)
            y = y * (1.5 - 0.5 * x * y * y)
            y = y * (1.5 - 0.5 * x * y * y)
            y = y * (1.5 - 0.5 * x * y * y)
            z = a + x * y
            zb = plsc.bitcast(z, jnp.int32)
            e = lax.shift_right_logical(zb, 23) - 127
            m = plsc.bitcast((zb & 0x007FFFFF) | 0x3F800000, jnp.float32)
            t = (m - 1.0) / (m + 1.0)
            t2 = t * t
            lnm = 2.0 * t * (1.0 + t2 * (1.0 / 3.0 + t2 * (0.2 + t2 * (1.0 / 7.0))))
            dist = LN2 * e.astype(jnp.float32) + lnm
            out_v[gsl] = gb + ubias_v[gsl] + ibias_v[gsl] + dist
            return 0

        lax.fori_loop(0, GPC, group_body, 0)

    inflight = {0: fire(0), 1: fire(1)}
    for c in range(NCHUNK):
        for cp in inflight.pop(c):
            cp.wait()
        compute_chunk(c)
        if c + 2 < NCHUNK:
            inflight[c + 2] = fire(c + 2)

    pltpu.sync_copy(out_v, out_hbm.at[pl.ds(base, BPW)])


_sc_kernel = functools.partial(
    pl.kernel,
    out_type=jax.ShapeDtypeStruct((BATCH,), jnp.float32),
    mesh=plsc.VectorSubcoreMesh(core_axis_name="c", subcore_axis_name="s"),
    compiler_params=pltpu.CompilerParams(
        needs_layout_passes=False, use_tc_tiling_on_sc=False
    ),
    scratch_types=[
        pltpu.VMEM((BPW,), jnp.int32),
        pltpu.VMEM((BPW,), jnp.int32),
        pltpu.VMEM((2, CHUNK, N_DIM), jnp.float32),
        pltpu.VMEM((2, CHUNK, N_DIM), jnp.float32),
        pltpu.VMEM((BPW,), jnp.float32),
        pltpu.VMEM((BPW,), jnp.float32),
        pltpu.VMEM((L,), jnp.float32),
        pltpu.VMEM((BPW,), jnp.float32),
        pltpu.SemaphoreType.DMA((NCHUNK,)),
    ],
)(_sc_body)


@jax.jit
def _impl(u, i, user_vect, user_bias, item_vect, item_bias, glob_bias):
    return _sc_kernel(
        u.astype(jnp.int32), i.astype(jnp.int32),
        user_vect, user_bias.reshape(-1),
        item_vect, item_bias.reshape(-1),
        jnp.broadcast_to(glob_bias.reshape(1), (L,)),
    )


def kernel(u, i, user_vect, user_bias, item_vect, item_bias, glob_bias):
    return _impl(u, i, user_vect, user_bias, item_vect, item_bias, glob_bias)
